# baseline (speedup 1.0000x reference)
import dataclasses
import functools

import jax
import jax.numpy as jnp
from jax import lax
from jax.experimental import pallas as pl
from jax.experimental.pallas import tpu as pltpu
from jax.experimental.pallas import tpu_sc as plsc

HIDDEN = 768
HALF = HIDDEN // 2
EPS = 1e-12

NC = 2
NS = 16
NW = NC * NS

TOKENS = 8192
N_SPLIT = 2
SPLIT = TOKENS // N_SPLIT
B_PER_W = SPLIT // NW
CHUNK = 32
N_CHUNKS = B_PER_W // CHUNK
L = 16

TOK_BLK = 2048


def _sc_gather_pack(table, ids2d, tok_base, seq):
    mesh = plsc.VectorSubcoreMesh(core_axis_name="c", subcore_axis_name="s")
    w_per_row = seq // B_PER_W
    cp = pltpu.CompilerParams()
    if "needs_layout_passes" in pltpu.CompilerParams.__dataclass_fields__:
        cp = dataclasses.replace(cp, needs_layout_passes=False)

    @functools.partial(
        pl.kernel,
        mesh=mesh,
        compiler_params=cp,
        out_type=jax.ShapeDtypeStruct((SPLIT, HALF), jnp.int32),
        scratch_types=[
            pltpu.VMEM((B_PER_W,), jnp.int32),
            pltpu.VMEM((CHUNK, HIDDEN), jnp.float32),
            pltpu.VMEM((CHUNK, HIDDEN), jnp.float32),
            pltpu.VMEM((CHUNK, HALF), jnp.int32),
            pltpu.VMEM((CHUNK, HALF), jnp.int32),
            pltpu.SemaphoreType.DMA,
            pltpu.SemaphoreType.DMA,
            pltpu.SemaphoreType.DMA,
            pltpu.SemaphoreType.DMA,
        ],
    )
    def k(table_hbm, idx_hbm, out_hbm, idx_v, f0, f1, p0, p1, g0, g1, s0, s1):
        wid = lax.axis_index("s") * NC + lax.axis_index("c")
        tok = tok_base + wid * B_PER_W
        pltpu.sync_copy(
            idx_hbm.at[tok // seq, pl.ds((wid % w_per_row) * B_PER_W, B_PER_W)],
            idx_v,
        )
        fbuf = (f0, f1)
        pbuf = (p0, p1)
        gsem = (g0, g1)
        ssem = (s0, s1)
        gathers = [None, None]
        stores = [None, None]
        for c in range(min(2, N_CHUNKS)):
            gathers[c] = pltpu.async_copy(
                table_hbm.at[idx_v.at[pl.ds(c * CHUNK, CHUNK)]], fbuf[c], gsem[c]
            )
        for c in range(N_CHUNKS):
            b = c % 2
            gathers[b].wait()
            if stores[b] is not None:
                stores[b].wait()

            @pl.loop(0, CHUNK)
            def _(r, _fb=fbuf[b], _pb=pbuf[b]):
                for j in range(HALF // L):
                    lo = _fb[r, pl.ds(j * L, L)]
                    hi = _fb[r, pl.ds(HALF + j * L, L)]
                    packed = plsc.pack(lo, hi, format=plsc.PackFormat.INTERLEAVED)
                    _pb[r, pl.ds(j * L, L)] = plsc.bitcast(packed, jnp.int32)

            stores[b] = pltpu.async_copy(
                pbuf[b],
                out_hbm.at[pl.ds(wid * B_PER_W + c * CHUNK, CHUNK)], ssem[b],
            )
            if c + 2 < N_CHUNKS:
                gathers[b] = pltpu.async_copy(
                    table_hbm.at[idx_v.at[pl.ds((c + 2) * CHUNK, CHUNK)]],
                    fbuf[b], gsem[b],
                )
        for st in stores:
            if st is not None:
                st.wait()

    return k(table, ids2d)


def _ln_math(x_ref, pos_ref, g_ref, b_ref, o_ref):
    xi = x_ref[...]
    lo = lax.bitcast_convert_type(jnp.left_shift(xi, 16), jnp.float32)
    hi = lax.bitcast_convert_type(
        jnp.bitwise_and(xi, jnp.int32(-65536)), jnp.float32
    )
    xl = lo + pos_ref[:, :HALF]
    xh = hi + pos_ref[:, HALF:]
    mean = (jnp.sum(xl, 1, keepdims=True) + jnp.sum(xh, 1, keepdims=True)) / HIDDEN
    cl = xl - mean
    ch = xh - mean
    var = (jnp.sum(cl * cl, 1, keepdims=True)
           + jnp.sum(ch * ch, 1, keepdims=True)) / HIDDEN
    inv = lax.rsqrt(var + EPS)
    o_ref[:, :HALF] = cl * inv * g_ref[:, :HALF] + b_ref[:, :HALF]
    o_ref[:, HALF:] = ch * inv * g_ref[:, HALF:] + b_ref[:, HALF:]


def _ln_body_first(x_ref, pos_ref, g_ref, b_ref, o_ref):
    _ln_math(x_ref, pos_ref, g_ref, b_ref, o_ref)


def _ln_body_alias(x_ref, pos_ref, g_ref, b_ref, prev_ref, o_ref):
    del prev_ref
    _ln_math(x_ref, pos_ref, g_ref, b_ref, o_ref)


def _tc_ln_half(gathered, pos, gamma, beta, batch_half, seq_len, blk_base, prev):
    bps = seq_len // TOK_BLK
    in_specs = [
        pl.BlockSpec((TOK_BLK, HALF), lambda i, j: (j * bps + i, 0)),
        pl.BlockSpec((TOK_BLK, HIDDEN), lambda i, j: (i, 0)),
        pl.BlockSpec((1, HIDDEN), lambda i, j: (0, 0)),
        pl.BlockSpec((1, HIDDEN), lambda i, j: (0, 0)),
    ]
    args = [gathered, pos, gamma.reshape(1, HIDDEN), beta.reshape(1, HIDDEN)]
    kwargs = {}
    if prev is None:
        body = _ln_body_first
    else:
        body = _ln_body_alias
        in_specs.append(pl.BlockSpec(memory_space=pl.ANY))
        args.append(prev)
        kwargs["input_output_aliases"] = {4: 0}
    return pl.pallas_call(
        body,
        grid=(bps, batch_half),
        in_specs=in_specs,
        out_specs=pl.BlockSpec(
            (TOK_BLK, HIDDEN), lambda i, j: (blk_base + j * bps + i, 0)
        ),
        out_shape=jax.ShapeDtypeStruct((TOKENS, HIDDEN), jnp.float32),
        **kwargs,
    )(*args)


def kernel(input_ids, word_embeddings, position_embeddings, ln_gamma, ln_beta):
    batch, seq = input_ids.shape
    assert batch * seq == TOKENS
    assert batch % N_SPLIT == 0 and seq % B_PER_W == 0
    batch_half = batch // N_SPLIT
    ids2d = input_ids.astype(jnp.int32)

    gathered = [
        _sc_gather_pack(word_embeddings, ids2d, h * SPLIT, seq)
        for h in range(N_SPLIT)
    ]
    out = None
    for h in range(N_SPLIT):
        out = _tc_ln_half(
            gathered[h], position_embeddings, ln_gamma, ln_beta,
            batch_half, seq, h * (SPLIT // TOK_BLK), out,
        )
    return out.reshape(batch, seq, HIDDEN)


# speedup vs baseline: 1.2156x; 1.2156x over previous
import dataclasses
import functools

import jax
import jax.numpy as jnp
from jax import lax
from jax.experimental import pallas as pl
from jax.experimental.pallas import tpu as pltpu
from jax.experimental.pallas import tpu_sc as plsc

HIDDEN = 768
HALF = HIDDEN // 2
EPS = 1e-12

NC = 2
NS = 16
NW = NC * NS

TOKENS = 8192
N_SPLIT = 2
SPLIT = TOKENS // N_SPLIT
B_PER_W = SPLIT // NW
CHUNK = 32
N_CHUNKS = B_PER_W // CHUNK
L = 16

TOK_BLK = 2048


def _sc_gather_pack(table, ids2d, tok_base, seq):
    mesh = plsc.VectorSubcoreMesh(core_axis_name="c", subcore_axis_name="s")
    w_per_row = seq // B_PER_W
    cp = pltpu.CompilerParams()
    if "needs_layout_passes" in pltpu.CompilerParams.__dataclass_fields__:
        cp = dataclasses.replace(cp, needs_layout_passes=False)

    @functools.partial(
        pl.kernel,
        mesh=mesh,
        compiler_params=cp,
        out_type=jax.ShapeDtypeStruct((SPLIT, HALF), jnp.int32),
        scratch_types=[
            pltpu.VMEM((B_PER_W,), jnp.int32),
            pltpu.VMEM((CHUNK, HIDDEN), jnp.float32),
            pltpu.VMEM((CHUNK, HIDDEN), jnp.float32),
            pltpu.VMEM((CHUNK, HALF), jnp.int32),
            pltpu.VMEM((CHUNK, HALF), jnp.int32),
            pltpu.SemaphoreType.DMA,
            pltpu.SemaphoreType.DMA,
            pltpu.SemaphoreType.DMA,
            pltpu.SemaphoreType.DMA,
        ],
    )
    def k(table_hbm, idx_hbm, out_hbm, idx_v, f0, f1, p0, p1, g0, g1, s0, s1):
        wid = lax.axis_index("s") * NC + lax.axis_index("c")
        tok = tok_base + wid * B_PER_W
        pltpu.sync_copy(
            idx_hbm.at[tok // seq, pl.ds((wid % w_per_row) * B_PER_W, B_PER_W)],
            idx_v,
        )
        fbuf = (f0, f1)
        pbuf = (p0, p1)
        gsem = (g0, g1)
        ssem = (s0, s1)
        gathers = [None, None]
        stores = [None, None]
        for c in range(min(2, N_CHUNKS)):
            gathers[c] = pltpu.async_copy(
                table_hbm.at[idx_v.at[pl.ds(c * CHUNK, CHUNK)]], fbuf[c], gsem[c]
            )
        for c in range(N_CHUNKS):
            b = c % 2
            gathers[b].wait()
            if stores[b] is not None:
                stores[b].wait()

            @plsc.parallel_loop(0, CHUNK, unroll=4)
            def _(r, _fb=fbuf[b], _pb=pbuf[b]):
                for j in range(HALF // L):
                    lo = _fb[r, pl.ds(j * L, L)]
                    hi = _fb[r, pl.ds(HALF + j * L, L)]
                    packed = plsc.pack(lo, hi, format=plsc.PackFormat.INTERLEAVED)
                    _pb[r, pl.ds(j * L, L)] = plsc.bitcast(packed, jnp.int32)

            stores[b] = pltpu.async_copy(
                pbuf[b],
                out_hbm.at[pl.ds(wid * B_PER_W + c * CHUNK, CHUNK)], ssem[b],
            )
            if c + 2 < N_CHUNKS:
                gathers[b] = pltpu.async_copy(
                    table_hbm.at[idx_v.at[pl.ds((c + 2) * CHUNK, CHUNK)]],
                    fbuf[b], gsem[b],
                )
        for st in stores:
            if st is not None:
                st.wait()

    return k(table, ids2d)


def _ln_math(x_ref, pos_ref, g_ref, b_ref, o_ref):
    xi = x_ref[...]
    lo = lax.bitcast_convert_type(jnp.left_shift(xi, 16), jnp.float32)
    hi = lax.bitcast_convert_type(
        jnp.bitwise_and(xi, jnp.int32(-65536)), jnp.float32
    )
    xl = lo + pos_ref[:, :HALF]
    xh = hi + pos_ref[:, HALF:]
    mean = (jnp.sum(xl, 1, keepdims=True) + jnp.sum(xh, 1, keepdims=True)) / HIDDEN
    cl = xl - mean
    ch = xh - mean
    var = (jnp.sum(cl * cl, 1, keepdims=True)
           + jnp.sum(ch * ch, 1, keepdims=True)) / HIDDEN
    inv = lax.rsqrt(var + EPS)
    o_ref[:, :HALF] = cl * inv * g_ref[:, :HALF] + b_ref[:, :HALF]
    o_ref[:, HALF:] = ch * inv * g_ref[:, HALF:] + b_ref[:, HALF:]


def _ln_body_first(x_ref, pos_ref, g_ref, b_ref, o_ref):
    _ln_math(x_ref, pos_ref, g_ref, b_ref, o_ref)


def _ln_body_alias(x_ref, pos_ref, g_ref, b_ref, prev_ref, o_ref):
    del prev_ref
    _ln_math(x_ref, pos_ref, g_ref, b_ref, o_ref)


def _tc_ln_half(gathered, pos, gamma, beta, batch_half, seq_len, blk_base, prev):
    bps = seq_len // TOK_BLK
    in_specs = [
        pl.BlockSpec((TOK_BLK, HALF), lambda i, j: (j * bps + i, 0)),
        pl.BlockSpec((TOK_BLK, HIDDEN), lambda i, j: (i, 0)),
        pl.BlockSpec((1, HIDDEN), lambda i, j: (0, 0)),
        pl.BlockSpec((1, HIDDEN), lambda i, j: (0, 0)),
    ]
    args = [gathered, pos, gamma.reshape(1, HIDDEN), beta.reshape(1, HIDDEN)]
    kwargs = {}
    if prev is None:
        body = _ln_body_first
    else:
        body = _ln_body_alias
        in_specs.append(pl.BlockSpec(memory_space=pl.ANY))
        args.append(prev)
        kwargs["input_output_aliases"] = {4: 0}
    return pl.pallas_call(
        body,
        grid=(bps, batch_half),
        in_specs=in_specs,
        out_specs=pl.BlockSpec(
            (TOK_BLK, HIDDEN), lambda i, j: (blk_base + j * bps + i, 0)
        ),
        out_shape=jax.ShapeDtypeStruct((TOKENS, HIDDEN), jnp.float32),
        **kwargs,
    )(*args)


def kernel(input_ids, word_embeddings, position_embeddings, ln_gamma, ln_beta):
    batch, seq = input_ids.shape
    assert batch * seq == TOKENS
    assert batch % N_SPLIT == 0 and seq % B_PER_W == 0
    batch_half = batch // N_SPLIT
    ids2d = input_ids.astype(jnp.int32)

    gathered = [
        _sc_gather_pack(word_embeddings, ids2d, h * SPLIT, seq)
        for h in range(N_SPLIT)
    ]
    out = None
    for h in range(N_SPLIT):
        out = _tc_ln_half(
            gathered[h], position_embeddings, ln_gamma, ln_beta,
            batch_half, seq, h * (SPLIT // TOK_BLK), out,
        )
    return out.reshape(batch, seq, HIDDEN)
